# baseline (speedup 1.0000x reference)
import jax
import jax.numpy as jnp
from jax import lax
from jax.experimental import pallas as pl
from jax.experimental.pallas import tpu as pltpu

B, S, D = 16, 32, 1024
E, TOPK, DFF = 8, 2, 1024
T = B * S
BLK = 128
NR = T * TOPK
NCH = NR // BLK + E
NPAD = NCH * BLK


def _routing_body(x_ref, r_ref, rt_ref, rw_ref, xg_ref, bexp_ref, tot_ref):
    x = x_ref[...]
    logits = lax.dot_general(x, r_ref[...], (((1,), (1,)), ((), ())),
                             preferred_element_type=jnp.float32)
    idx = lax.broadcasted_iota(jnp.int32, (T, E), 1)
    m0 = jnp.max(logits, axis=1, keepdims=True)
    i0 = jnp.min(jnp.where(logits == m0, idx, E), axis=1, keepdims=True)
    masked = jnp.where(idx == i0, -jnp.inf, logits)
    m1 = jnp.max(masked, axis=1, keepdims=True)
    i1 = jnp.min(jnp.where(masked == m1, idx, E), axis=1, keepdims=True)
    e1 = jnp.exp(m1 - m0)
    denom = 1.0 + e1
    w0 = 1.0 / denom
    w1 = e1 / denom
    s = w0 + w1
    w0 = w0 / s
    w1 = w1 / s

    tril = (lax.broadcasted_iota(jnp.int32, (T, T), 0)
            > lax.broadcasted_iota(jnp.int32, (T, T), 1)).astype(jnp.float32)
    pos0 = jnp.zeros((T, 1), jnp.float32)
    pos1 = jnp.zeros((T, 1), jnp.float32)
    offblk = jnp.int32(0)
    offs, nbs = [], []
    emax = jnp.int32(0)
    for e in range(E):
        m0e = (i0 == e)
        m1e = (i1 == e)
        f0 = m0e.astype(jnp.float32)
        f1 = m1e.astype(jnp.float32)
        cnt_t = f0 + f1
        pre = lax.dot_general(tril, cnt_t, (((1,), (0,)), ((), ())),
                              preferred_element_type=jnp.float32)
        count_e = jnp.sum(cnt_t).astype(jnp.int32)
        nb_e = (count_e + BLK - 1) // BLK
        base = (offblk * BLK).astype(jnp.float32)
        pos0 = pos0 + jnp.where(m0e, base + pre, 0.0)
        pos1 = pos1 + jnp.where(m1e, base + pre + f0, 0.0)
        offs.append(offblk)
        nbs.append(nb_e)
        emax = jnp.where(nb_e > 0, jnp.int32(e), emax)
        offblk = offblk + nb_e

    tot_ref[0, 0] = offblk
    for b in range(NCH):
        bexp_b = emax
        for e in range(E):
            inside = (b >= offs[e]) & (b < offs[e] + nbs[e])
            bexp_b = jnp.where(inside, jnp.int32(e), bexp_b)
        bexp_ref[0, b] = bexp_b

    lane_r = lax.broadcasted_iota(jnp.int32, (T, NPAD), 1).astype(jnp.float32)
    m0t = jnp.where(lane_r == pos0, 1.0, 0.0)
    m1t = jnp.where(lane_r == pos1, 1.0, 0.0)
    t_col = lax.broadcasted_iota(jnp.int32, (T, 1), 0).astype(jnp.float32)
    rhs0 = jnp.concatenate([t_col, w0], axis=1)
    rhs1 = jnp.concatenate([t_col, w1], axis=1)
    out0 = lax.dot_general(m0t, rhs0, (((0,), (0,)), ((), ())),
                           preferred_element_type=jnp.float32,
                           precision=lax.Precision.HIGHEST)
    out1 = lax.dot_general(m1t, rhs1, (((0,), (0,)), ((), ())),
                           preferred_element_type=jnp.float32,
                           precision=lax.Precision.HIGHEST)
    rt_ref[...] = out0[:, 0:1] + out1[:, 0:1]
    rw_ref[...] = out0[:, 1:2] + out1[:, 1:2]
    xg_ref[...] = lax.dot_general(m0t + m1t, x, (((0,), (0,)), ((), ())),
                                  preferred_element_type=jnp.float32)


@jax.jit
def _routing(x, router):
    return pl.pallas_call(
        _routing_body,
        in_specs=[
            pl.BlockSpec((T, D), lambda: (0, 0)),
            pl.BlockSpec((E, D), lambda: (0, 0)),
        ],
        out_specs=[
            pl.BlockSpec((NPAD, 1), lambda: (0, 0)),
            pl.BlockSpec((NPAD, 1), lambda: (0, 0)),
            pl.BlockSpec((NPAD, D), lambda: (0, 0)),
            pl.BlockSpec((1, NCH), lambda: (0, 0), memory_space=pltpu.SMEM),
            pl.BlockSpec((1, 1), lambda: (0, 0), memory_space=pltpu.SMEM),
        ],
        out_shape=[
            jax.ShapeDtypeStruct((NPAD, 1), jnp.float32),
            jax.ShapeDtypeStruct((NPAD, 1), jnp.float32),
            jax.ShapeDtypeStruct((NPAD, D), jnp.float32),
            jax.ShapeDtypeStruct((1, NCH), jnp.int32),
            jax.ShapeDtypeStruct((1, 1), jnp.int32),
        ],
    )(x, router)


def _mlp_body(bexp_ref, tot_ref, xg_ref, g_ref, u_ref, d_ref, rt_ref, rw_ref,
              o_ref):
    b = pl.program_id(0)

    @pl.when(b == 0)
    def _init():
        o_ref[...] = jnp.zeros_like(o_ref)

    @pl.when(b < tot_ref[0])
    def _compute():
        tok = rt_ref[...]
        w = rw_ref[...]
        xrows = xg_ref[...]
        g = lax.dot_general(xrows, g_ref[0], (((1,), (1,)), ((), ())),
                            preferred_element_type=jnp.float32)
        u = lax.dot_general(xrows, u_ref[0], (((1,), (1,)), ((), ())),
                            preferred_element_type=jnp.float32)
        inter = g * lax.logistic(g) * u
        eo = lax.dot_general(inter, d_ref[0], (((1,), (1,)), ((), ())),
                             preferred_element_type=jnp.float32)
        lane_t = lax.broadcasted_iota(jnp.int32, (BLK, T), 1).astype(jnp.float32)
        c = jnp.where(lane_t == tok, w, 0.0)
        o_ref[...] += lax.dot_general(c, eo, (((0,), (0,)), ((), ())),
                                      preferred_element_type=jnp.float32)


@jax.jit
def _mlp(bexp, tot, xg, gate_proj, up_proj, down_proj, rt, rw):
    grid_spec = pltpu.PrefetchScalarGridSpec(
        num_scalar_prefetch=2,
        grid=(NCH,),
        in_specs=[
            pl.BlockSpec((BLK, D), lambda b, bexp, tot: (b, 0)),
            pl.BlockSpec((1, DFF, D), lambda b, bexp, tot: (bexp[b], 0, 0)),
            pl.BlockSpec((1, DFF, D), lambda b, bexp, tot: (bexp[b], 0, 0)),
            pl.BlockSpec((1, D, DFF), lambda b, bexp, tot: (bexp[b], 0, 0)),
            pl.BlockSpec((BLK, 1), lambda b, bexp, tot: (b, 0)),
            pl.BlockSpec((BLK, 1), lambda b, bexp, tot: (b, 0)),
        ],
        out_specs=pl.BlockSpec((T, D), lambda b, bexp, tot: (0, 0)),
    )
    return pl.pallas_call(
        _mlp_body,
        grid_spec=grid_spec,
        out_shape=jax.ShapeDtypeStruct((T, D), jnp.float32),
    )(bexp, tot, xg, gate_proj, up_proj, down_proj, rt, rw)


def kernel(hidden_states, router, gate_proj, up_proj, down_proj):
    b, s, d = hidden_states.shape
    x = hidden_states.reshape(-1, d)
    rt, rw, xg, bexp, tot = _routing(x, router)
    out = _mlp(bexp.reshape(NCH), tot.reshape(1), xg,
               gate_proj, up_proj, down_proj, rt, rw)
    return out.reshape(b, s, d)


# speedup vs baseline: 1.6682x; 1.6682x over previous
import functools

import jax
import jax.numpy as jnp
from jax.experimental import pallas as pl
from jax.experimental.pallas import tpu as pltpu

B, S, D = 16, 32, 1024
E, TOPK, DFF = 8, 2, 1024
T = B * S


def _moe_dense_body(x_ref, r_ref, g_ref, u_ref, d_ref, o_ref, w_ref):
    e = pl.program_id(0)

    @pl.when(e == 0)
    def _route():
        x = x_ref[...]
        logits = jax.lax.dot_general(
            x, r_ref[...], (((1,), (1,)), ((), ())),
            preferred_element_type=jnp.float32)
        idx = jax.lax.broadcasted_iota(jnp.int32, (T, E), 1)
        m0 = jnp.max(logits, axis=1, keepdims=True)
        i0 = jnp.min(jnp.where(logits == m0, idx, E), axis=1, keepdims=True)
        masked = jnp.where(idx == i0, -jnp.inf, logits)
        m1 = jnp.max(masked, axis=1, keepdims=True)
        i1 = jnp.min(jnp.where(masked == m1, idx, E), axis=1, keepdims=True)
        e1 = jnp.exp(m1 - m0)
        denom = 1.0 + e1
        w0 = 1.0 / denom
        w1 = e1 / denom
        wsum = w0 + w1
        w0 = w0 / wsum
        w1 = w1 / wsum
        w_ref[...] = jnp.where(idx == i0, w0, 0.0) + jnp.where(idx == i1, w1, 0.0)

    x = x_ref[...]
    gate = jax.lax.dot_general(
        x, g_ref[0], (((1,), (1,)), ((), ())), preferred_element_type=jnp.float32)
    up = jax.lax.dot_general(
        x, u_ref[0], (((1,), (1,)), ((), ())), preferred_element_type=jnp.float32)
    inter = gate * jax.lax.logistic(gate) * up
    eo = jax.lax.dot_general(
        inter, d_ref[0], (((1,), (1,)), ((), ())), preferred_element_type=jnp.float32)
    idx = jax.lax.broadcasted_iota(jnp.int32, (T, E), 1)
    wcol = jnp.sum(w_ref[...] * jnp.where(idx == e, 1.0, 0.0), axis=1, keepdims=True)
    contrib = eo * wcol

    @pl.when(e == 0)
    def _init():
        o_ref[...] = contrib

    @pl.when(e > 0)
    def _acc():
        o_ref[...] += contrib


@jax.jit
def _moe_dense(x, router, gate_proj, up_proj, down_proj):
    out = pl.pallas_call(
        _moe_dense_body,
        grid=(E,),
        in_specs=[
            pl.BlockSpec((T, D), lambda e: (0, 0)),
            pl.BlockSpec((E, D), lambda e: (0, 0)),
            pl.BlockSpec((1, DFF, D), lambda e: (e, 0, 0)),
            pl.BlockSpec((1, DFF, D), lambda e: (e, 0, 0)),
            pl.BlockSpec((1, D, DFF), lambda e: (e, 0, 0)),
        ],
        out_specs=pl.BlockSpec((T, D), lambda e: (0, 0)),
        out_shape=jax.ShapeDtypeStruct((T, D), jnp.float32),
        scratch_shapes=[pltpu.VMEM((T, E), jnp.float32)],
    )(x, router, gate_proj, up_proj, down_proj)
    return out


def kernel(hidden_states, router, gate_proj, up_proj, down_proj):
    b, s, d = hidden_states.shape
    x = hidden_states.reshape(-1, d)
    out = _moe_dense(x, router, gate_proj, up_proj, down_proj)
    return out.reshape(b, s, d)
